# baseline (speedup 1.0000x reference)
import functools

import jax
import jax.numpy as jnp
from jax import lax
from jax.experimental import pallas as pl
from jax.experimental.pallas import tpu as pltpu
from jax.experimental.pallas import tpu_sc as plsc

NC = 2
NS = 16
NW = NC * NS
L = 16
C = 128

_F32 = jnp.float32
_I32 = jnp.int32


def _round_up(a, b):
    return (a + b - 1) // b * b


def _sc_mesh():
    return plsc.VectorSubcoreMesh(core_axis_name="c", subcore_axis_name="s")


def _wid():
    return lax.axis_index("c") * NS + lax.axis_index("s")


def _splat16(s):
    return jnp.broadcast_to(s, (L,)).astype(_I32)


def _make_deg_partial(npad, nch):
    @functools.partial(
        pl.kernel,
        out_type=jax.ShapeDtypeStruct((NW * npad,), _F32),
        mesh=_sc_mesh(),
        compiler_params=pltpu.CompilerParams(needs_layout_passes=False),
        scratch_types=[
            pltpu.VMEM((nch, C), _I32),
            pltpu.VMEM((nch, C), _F32),
            pltpu.VMEM((npad,), _F32),
        ],
    )
    def deg_partial(src_hbm, w_hbm, part_hbm, srcv, wv, partv):
        w = _wid()
        zz = jnp.zeros((L,), _F32)

        def zero(i, _):
            partv[pl.ds(i * L, L)] = zz
            return 0

        lax.fori_loop(0, npad // L, zero, 0)
        pltpu.sync_copy(src_hbm.at[w], srcv)
        pltpu.sync_copy(w_hbm.at[w], wv)

        def chunk(i, _):
            def grp(g, _2):
                idx = srcv[i, pl.ds(g * L, L)]
                val = wv[i, pl.ds(g * L, L)]
                plsc.addupdate_scatter(partv, [idx], val)
                return 0

            lax.fori_loop(0, C // L, grp, 0)
            return 0

        lax.fori_loop(0, nch, chunk, 0)
        pltpu.sync_copy(partv, part_hbm.at[pl.ds(pl.multiple_of(w * npad, 128), npad)])

    return deg_partial


def _make_dis(npad):
    m = npad // NW

    @functools.partial(
        pl.kernel,
        out_type=jax.ShapeDtypeStruct((npad,), _F32),
        mesh=_sc_mesh(),
        compiler_params=pltpu.CompilerParams(needs_layout_passes=False),
        scratch_types=[
            pltpu.VMEM((NW * m,), _F32),
            pltpu.VMEM((m,), _F32),
        ],
    )
    def dis_kernel(part_hbm, dis_hbm, bufv, disv):
        w = _wid()
        base = pl.multiple_of(w * m, 8)
        for p in range(NW):
            pltpu.sync_copy(part_hbm.at[pl.ds(base + p * npad, m)],
                            bufv.at[pl.ds(p * m, m)])

        def col(j, _):
            def acc(p, v):
                return v + bufv[pl.ds(p * m + j * L, L)]

            s = lax.fori_loop(0, NW, acc, jnp.zeros((L,), _F32))
            sx = jnp.maximum(s, jnp.float32(1e-30))
            ii = plsc.bitcast(sx, _I32)
            yi = jnp.int32(0x5F3759DF) - lax.shift_right_logical(ii, 1)
            y = plsc.bitcast(yi, _F32)
            for _ in range(4):
                y = y * (jnp.float32(1.5) - jnp.float32(0.5) * sx * y * y)
            disv[pl.ds(j * L, L)] = jnp.where(s > 0.0, y, jnp.float32(0.0))
            return 0

        lax.fori_loop(0, m // L, col, 0)
        pltpu.sync_copy(disv, dis_hbm.at[pl.ds(base, m)])

    return dis_kernel


def _make_lw(npad, nch):
    @functools.partial(
        pl.kernel,
        out_type=jax.ShapeDtypeStruct((NW * nch * C,), _F32),
        mesh=_sc_mesh(),
        compiler_params=pltpu.CompilerParams(needs_layout_passes=False),
        scratch_types=[
            pltpu.VMEM((nch, C), _I32),
            pltpu.VMEM((nch, C), _I32),
            pltpu.VMEM((nch, C), _F32),
            pltpu.VMEM((npad,), _F32),
            pltpu.VMEM((nch * C,), _F32),
        ],
    )
    def lw_kernel(src_hbm, dst_hbm, w_hbm, dis_hbm, lw_hbm, srcv, dstv, wv, disv, lwv):
        w = _wid()
        pltpu.sync_copy(dis_hbm, disv)
        pltpu.sync_copy(src_hbm.at[w], srcv)
        pltpu.sync_copy(dst_hbm.at[w], dstv)
        pltpu.sync_copy(w_hbm.at[w], wv)

        def chunk(i, _):
            def grp(g, _2):
                s16 = srcv[i, pl.ds(g * L, L)]
                d16 = dstv[i, pl.ds(g * L, L)]
                w16 = wv[i, pl.ds(g * L, L)]
                a = plsc.load_gather(disv, [s16])
                b = plsc.load_gather(disv, [d16])
                lwv[pl.ds(i * C + g * L, L)] = -(a * w16 * b)
                return 0

            lax.fori_loop(0, C // L, grp, 0)
            return 0

        lax.fori_loop(0, nch, chunk, 0)
        pltpu.sync_copy(lwv, lw_hbm.at[pl.ds(pl.multiple_of(w * nch * C, 128), nch * C)])

    return lw_kernel


def _make_prop(npad, nch0, nch1):
    rows_per_tile = npad // NS
    assert nch0 % 4 == 0 and nch1 % 4 == 0

    @functools.partial(
        pl.kernel,
        out_type=jax.ShapeDtypeStruct((NC, npad, 128), _F32),
        mesh=_sc_mesh(),
        compiler_params=pltpu.CompilerParams(needs_layout_passes=False),
        scratch_types=[
            pltpu.VMEM((max(nch0, nch1) * C,), _F32),
            pltpu.VMEM((C, 128), _F32),
            pltpu.VMEM((C, 128), _F32),
            pltpu.VMEM((C,), _I32),
            pltpu.VMEM((C,), _I32),
            pltpu.VMEM((C,), _I32),
            pltpu.VMEM((C,), _I32),
            pltpu.VMEM((C,), _I32),
            pltpu.VMEM((C,), _I32),
            pltpu.VMEM((C,), _I32),
            pltpu.VMEM((C,), _I32),
            pltpu.SemaphoreType.DMA,
            pltpu.SemaphoreType.DMA,
            pltpu.SemaphoreType.DMA,
            pltpu.SemaphoreType.DMA,
            pltpu.SemaphoreType.DMA,
            pltpu.SemaphoreType.DMA,
            pltpu.VMEM_SHARED((npad, 128), _F32),
        ],
    )
    def prop(v_hbm, src_hbm, dst_hbm, lw_hbm, part_hbm, lwv,
             r0, r1, sa0, sa1, sa2, sa3, da0, da1, da2, da3,
             g0, g1, i0, i1, i2, i3, acc):
        rows = (r0, r1)
        srcb = (sa0, sa1, sa2, sa3)
        dstb = (da0, da1, da2, da3)
        gs = (g0, g1)
        isem = (i0, i1, i2, i3)
        cid = lax.axis_index("c")
        sid = lax.axis_index("s")
        nch = jnp.where(cid == 0, nch0, nch1)
        base = jnp.where(cid == 0, sid * nch0, NS * nch0 + sid * nch1)
        zz = jnp.zeros((L,), _F32)

        with jax.named_scope("zero_acc"):
            def zrow(j, _):
                for f in range(128 // L):
                    r0[j, pl.ds(f * L, L)] = zz
                return 0

            lax.fori_loop(0, C, zrow, 0)

            def zacc(r, _):
                pltpu.sync_copy(r0, acc.at[pl.ds(sid * rows_per_tile + r * C, C)])
                return 0

            lax.fori_loop(0, rows_per_tile // C, zacc, 0)
            plsc.subcore_barrier()

        @pl.when(cid == 0)
        def _():
            pltpu.sync_copy(
                lw_hbm.at[pl.ds(pl.multiple_of(base * C, 128), nch0 * C)],
                lwv.at[pl.ds(0, nch0 * C)])

        @pl.when(cid == 1)
        def _():
            pltpu.sync_copy(
                lw_hbm.at[pl.ds(pl.multiple_of(base * C, 128), nch1 * C)],
                lwv.at[pl.ds(0, nch1 * C)])

        def eoff(i):
            return pl.ds(pl.multiple_of((base + i) * C, 128), C)

        def fire_idx(i, q):
            pltpu.async_copy(src_hbm.at[eoff(i)], srcb[q], isem[q])
            pltpu.async_copy(dst_hbm.at[eoff(i)], dstb[q], isem[q])

        def wait_idx(q):
            pltpu.make_async_copy(src_hbm.at[pl.ds(0, C)], srcb[q], isem[q]).wait()
            pltpu.make_async_copy(dst_hbm.at[pl.ds(0, C)], dstb[q], isem[q]).wait()

        def fire_gather(b, q):
            pltpu.async_copy(v_hbm.at[srcb[q]], rows[b], gs[b])

        def wait_gather(b):
            pltpu.make_async_copy(v_hbm.at[srcb[0]], rows[b], gs[b]).wait()

        def scale(i, b):
            rb = rows[b]

            @plsc.parallel_loop(0, C, 1, unroll=4)
            def _(j):
                w16 = plsc.load_gather(lwv, [_splat16(i * C + j)])
                for f in range(128 // L):
                    rb[j, pl.ds(f * L, L)] = rb[j, pl.ds(f * L, L)] * w16

        with jax.named_scope("edges"):
            pltpu.sync_copy(src_hbm.at[eoff(0)], sa0)
            pltpu.sync_copy(dst_hbm.at[eoff(0)], da0)
            fire_idx(1, 1)
            fire_gather(0, 0)

            def quad(t, _):
                for b4 in range(4):
                    i = t * 4 + b4
                    b = b4 % 2

                    @pl.when(i + 1 < nch)
                    def _():
                        wait_idx((b4 + 1) % 4)
                        fire_gather(1 - b, (b4 + 1) % 4)

                    wait_gather(b)

                    @pl.when(i + 2 < nch)
                    def _():
                        fire_idx(i + 2, (b4 + 2) % 4)

                    scale(i, b)
                    pltpu.sync_copy(rows[b], acc.at[dstb[b4]], add=True)
                return 0

            lax.fori_loop(0, nch // 4, quad, 0)
            plsc.subcore_barrier()

        with jax.named_scope("drain"):
            def drain(r, _):
                off = sid * rows_per_tile + r * C
                pltpu.sync_copy(acc.at[pl.ds(off, C)], part_hbm.at[cid, pl.ds(off, C)])
                return 0

            lax.fori_loop(0, rows_per_tile // C, drain, 0)

    return prop


def _combine(parts, prev):
    npad = parts.shape[1]
    R = 512

    def body(p_ref, q_ref, o_ref):
        o_ref[...] = 2.0 * (p_ref[0] + p_ref[1]) - q_ref[...]

    return pl.pallas_call(
        body,
        grid=(npad // R,),
        in_specs=[
            pl.BlockSpec((NC, R, 128), lambda i: (0, i, 0)),
            pl.BlockSpec((R, 128), lambda i: (i, 0)),
        ],
        out_specs=pl.BlockSpec((R, 128), lambda i: (i, 0)),
        out_shape=jax.ShapeDtypeStruct((npad, 128), _F32),
    )(parts, prev)


def _combine1(parts):
    npad = parts.shape[1]
    R = 512

    def body(p_ref, o_ref):
        o_ref[...] = p_ref[0] + p_ref[1]

    return pl.pallas_call(
        body,
        grid=(npad // R,),
        in_specs=[pl.BlockSpec((NC, R, 128), lambda i: (0, i, 0))],
        out_specs=pl.BlockSpec((R, 128), lambda i: (i, 0)),
        out_shape=jax.ShapeDtypeStruct((npad, 128), _F32),
    )(parts)


def _final(xp, t1, t2, p3, Wc, bc, lin_W, lin_b):
    npad = xp.shape[0]
    R = 512

    def body(x_ref, t1_ref, t2_ref, p3_ref, w_ref, b_ref, lw_ref, lb_ref, o_ref):
        t1v = t1_ref[...]
        t3 = 2.0 * (p3_ref[0] + p3_ref[1]) - t1v
        a = jnp.dot(x_ref[...], w_ref[0], preferred_element_type=_F32)
        a = a + jnp.dot(t1v, w_ref[1], preferred_element_type=_F32)
        a = a + jnp.dot(t2_ref[...], w_ref[2], preferred_element_type=_F32)
        a = a + jnp.dot(t3, w_ref[3], preferred_element_type=_F32)
        a = a + b_ref[...]
        z = jax.nn.sigmoid(a[:, :128])
        ht = jnp.tanh(a[:, 128:])
        h = jax.nn.relu((1.0 - z) * ht)
        o_ref[...] = jnp.dot(h, lw_ref[...], preferred_element_type=_F32) + lb_ref[...]

    return pl.pallas_call(
        body,
        grid=(npad // R,),
        in_specs=[
            pl.BlockSpec((R, 128), lambda i: (i, 0)),
            pl.BlockSpec((R, 128), lambda i: (i, 0)),
            pl.BlockSpec((R, 128), lambda i: (i, 0)),
            pl.BlockSpec((NC, R, 128), lambda i: (0, i, 0)),
            pl.BlockSpec((4, 128, 256), lambda i: (0, 0, 0)),
            pl.BlockSpec((1, 256), lambda i: (0, 0)),
            pl.BlockSpec((128, 1), lambda i: (0, 0)),
            pl.BlockSpec((1, 1), lambda i: (0, 0)),
        ],
        out_specs=pl.BlockSpec((R, 1), lambda i: (i, 0)),
        out_shape=jax.ShapeDtypeStruct((npad, 1), _F32),
    )(xp, t1, t2, p3, Wc, bc, lin_W, lin_b)


def kernel(x, edge_index, edge_weight, W_xz, b_xz, W_hz, b_hz, W_xr, b_xr,
           W_hr, b_hr, W_xh, b_xh, W_hh, b_hh, lin_W, lin_b):
    n, d = x.shape
    e = edge_weight.shape[0]
    npad = _round_up(n, NS * C)
    epad = _round_up(e, NW * C * 4)
    nch = epad // (NW * C)

    src = jnp.zeros((epad,), _I32).at[:e].set(edge_index[0])
    dst = jnp.zeros((epad,), _I32).at[:e].set(edge_index[1])
    src3 = src.reshape(NW, nch, C)
    dst3 = dst.reshape(NW, nch, C)
    wgt = jnp.zeros((epad,), _F32).at[:e].set(edge_weight).reshape(NW, nch, C)
    xp = jnp.zeros((npad, d), _F32).at[:n].set(x)

    part_deg = _make_deg_partial(npad, nch)(src3, wgt)
    dis = _make_dis(npad)(part_deg)
    lw = _make_lw(npad, nch)(src3, dst3, wgt, dis)

    per_pair = epad // (NS * C)
    nch0 = (per_pair // 2) // 4 * 4
    nch1 = per_pair - nch0
    prop = _make_prop(npad, nch0, nch1)
    p1 = prop(xp, src, dst, lw)
    t1 = _combine1(p1)
    p2 = prop(t1, src, dst, lw)
    t2 = _combine(p2, xp)
    p3 = prop(t2, src, dst, lw)

    Wc = jnp.concatenate([W_xz, W_xh], axis=2)
    bc = jnp.concatenate([b_xz + b_hz, b_xh + b_hh]).reshape(1, 256)
    out = _final(xp, t1, t2, p3, Wc, bc, lin_W, lin_b.reshape(1, 1))
    return out[:n]


# speedup vs baseline: 1.0823x; 1.0823x over previous
import functools

import jax
import jax.numpy as jnp
from jax import lax
from jax.experimental import pallas as pl
from jax.experimental.pallas import tpu as pltpu
from jax.experimental.pallas import tpu_sc as plsc

NC = 2
NS = 16
NW = NC * NS
L = 16
C = 64

_F32 = jnp.float32
_I32 = jnp.int32


def _round_up(a, b):
    return (a + b - 1) // b * b


def _sc_mesh():
    return plsc.VectorSubcoreMesh(core_axis_name="c", subcore_axis_name="s")


def _wid():
    return lax.axis_index("c") * NS + lax.axis_index("s")


def _splat16(s):
    return jnp.broadcast_to(s, (L,)).astype(_I32)


def _make_deg_partial(npad, nch):
    @functools.partial(
        pl.kernel,
        out_type=jax.ShapeDtypeStruct((NW * npad,), _F32),
        mesh=_sc_mesh(),
        compiler_params=pltpu.CompilerParams(needs_layout_passes=False),
        scratch_types=[
            pltpu.VMEM((nch, C), _I32),
            pltpu.VMEM((nch, C), _F32),
            pltpu.VMEM((npad,), _F32),
        ],
    )
    def deg_partial(src_hbm, w_hbm, part_hbm, srcv, wv, partv):
        w = _wid()
        zz = jnp.zeros((L,), _F32)

        def zero(i, _):
            partv[pl.ds(i * L, L)] = zz
            return 0

        lax.fori_loop(0, npad // L, zero, 0)
        pltpu.sync_copy(src_hbm.at[w], srcv)
        pltpu.sync_copy(w_hbm.at[w], wv)

        def chunk(i, _):
            def grp(g, _2):
                idx = srcv[i, pl.ds(g * L, L)]
                val = wv[i, pl.ds(g * L, L)]
                plsc.addupdate_scatter(partv, [idx], val)
                return 0

            lax.fori_loop(0, C // L, grp, 0)
            return 0

        lax.fori_loop(0, nch, chunk, 0)
        pltpu.sync_copy(partv, part_hbm.at[pl.ds(pl.multiple_of(w * npad, 128), npad)])

    return deg_partial


def _make_dis(npad):
    m = npad // NW

    @functools.partial(
        pl.kernel,
        out_type=jax.ShapeDtypeStruct((npad,), _F32),
        mesh=_sc_mesh(),
        compiler_params=pltpu.CompilerParams(needs_layout_passes=False),
        scratch_types=[
            pltpu.VMEM((NW * m,), _F32),
            pltpu.VMEM((m,), _F32),
        ],
    )
    def dis_kernel(part_hbm, dis_hbm, bufv, disv):
        w = _wid()
        base = pl.multiple_of(w * m, 8)
        for p in range(NW):
            pltpu.sync_copy(part_hbm.at[pl.ds(base + p * npad, m)],
                            bufv.at[pl.ds(p * m, m)])

        def col(j, _):
            def acc(p, v):
                return v + bufv[pl.ds(p * m + j * L, L)]

            s = lax.fori_loop(0, NW, acc, jnp.zeros((L,), _F32))
            sx = jnp.maximum(s, jnp.float32(1e-30))
            ii = plsc.bitcast(sx, _I32)
            yi = jnp.int32(0x5F3759DF) - lax.shift_right_logical(ii, 1)
            y = plsc.bitcast(yi, _F32)
            for _ in range(4):
                y = y * (jnp.float32(1.5) - jnp.float32(0.5) * sx * y * y)
            disv[pl.ds(j * L, L)] = jnp.where(s > 0.0, y, jnp.float32(0.0))
            return 0

        lax.fori_loop(0, m // L, col, 0)
        pltpu.sync_copy(disv, dis_hbm.at[pl.ds(base, m)])

    return dis_kernel


def _make_lw(npad, nch):
    @functools.partial(
        pl.kernel,
        out_type=jax.ShapeDtypeStruct((NW * nch * C,), _F32),
        mesh=_sc_mesh(),
        compiler_params=pltpu.CompilerParams(needs_layout_passes=False),
        scratch_types=[
            pltpu.VMEM((nch, C), _I32),
            pltpu.VMEM((nch, C), _I32),
            pltpu.VMEM((nch, C), _F32),
            pltpu.VMEM((npad,), _F32),
            pltpu.VMEM((nch * C,), _F32),
        ],
    )
    def lw_kernel(src_hbm, dst_hbm, w_hbm, dis_hbm, lw_hbm, srcv, dstv, wv, disv, lwv):
        w = _wid()
        pltpu.sync_copy(dis_hbm, disv)
        pltpu.sync_copy(src_hbm.at[w], srcv)
        pltpu.sync_copy(dst_hbm.at[w], dstv)
        pltpu.sync_copy(w_hbm.at[w], wv)

        def chunk(i, _):
            def grp(g, _2):
                s16 = srcv[i, pl.ds(g * L, L)]
                d16 = dstv[i, pl.ds(g * L, L)]
                w16 = wv[i, pl.ds(g * L, L)]
                a = plsc.load_gather(disv, [s16])
                b = plsc.load_gather(disv, [d16])
                lwv[pl.ds(i * C + g * L, L)] = -(a * w16 * b)
                return 0

            lax.fori_loop(0, C // L, grp, 0)
            return 0

        lax.fori_loop(0, nch, chunk, 0)
        pltpu.sync_copy(lwv, lw_hbm.at[pl.ds(pl.multiple_of(w * nch * C, 128), nch * C)])

    return lw_kernel


def _make_prop(npad, nch0, nch1):
    rows_per_tile = npad // NS
    assert nch0 % 4 == 0 and nch1 % 4 == 0

    @functools.partial(
        pl.kernel,
        out_type=jax.ShapeDtypeStruct((NC, npad, 128), _F32),
        mesh=_sc_mesh(),
        compiler_params=pltpu.CompilerParams(needs_layout_passes=False),
        scratch_types=[
            pltpu.VMEM((max(nch0, nch1) * C,), _F32),
            pltpu.VMEM((C, 128), _F32),
            pltpu.VMEM((C, 128), _F32),
            pltpu.VMEM((C, 128), _F32),
            pltpu.VMEM((C, 128), _F32),
            pltpu.VMEM((C,), _I32),
            pltpu.VMEM((C,), _I32),
            pltpu.VMEM((C,), _I32),
            pltpu.VMEM((C,), _I32),
            pltpu.VMEM((C,), _I32),
            pltpu.VMEM((C,), _I32),
            pltpu.VMEM((C,), _I32),
            pltpu.VMEM((C,), _I32),
            pltpu.SemaphoreType.DMA,
            pltpu.SemaphoreType.DMA,
            pltpu.SemaphoreType.DMA,
            pltpu.SemaphoreType.DMA,
            pltpu.SemaphoreType.DMA,
            pltpu.SemaphoreType.DMA,
            pltpu.SemaphoreType.DMA,
            pltpu.SemaphoreType.DMA,
            pltpu.SemaphoreType.DMA,
            pltpu.SemaphoreType.DMA,
            pltpu.SemaphoreType.DMA,
            pltpu.SemaphoreType.DMA,
            pltpu.VMEM_SHARED((npad, 128), _F32),
        ],
    )
    def prop(v_hbm, src_hbm, dst_hbm, lw_hbm, part_hbm, lwv,
             r0, r1, r2, r3, sa0, sa1, sa2, sa3, da0, da1, da2, da3,
             g0, g1, g2, g3, i0, i1, i2, i3, s0, s1, s2, s3, acc):
        rows = (r0, r1, r2, r3)
        srcb = (sa0, sa1, sa2, sa3)
        dstb = (da0, da1, da2, da3)
        gs = (g0, g1, g2, g3)
        isem = (i0, i1, i2, i3)
        ssem = (s0, s1, s2, s3)
        cid = lax.axis_index("c")
        sid = lax.axis_index("s")
        nch = jnp.where(cid == 0, nch0, nch1)
        base = jnp.where(cid == 0, sid * nch0, NS * nch0 + sid * nch1)
        zz = jnp.zeros((L,), _F32)

        with jax.named_scope("zero_acc"):
            def zrow(j, _):
                for f in range(128 // L):
                    r0[j, pl.ds(f * L, L)] = zz
                return 0

            lax.fori_loop(0, C, zrow, 0)

            def zacc(r, _):
                pltpu.sync_copy(r0, acc.at[pl.ds(sid * rows_per_tile + r * C, C)])
                return 0

            lax.fori_loop(0, rows_per_tile // C, zacc, 0)
            plsc.subcore_barrier()

        @pl.when(cid == 0)
        def _():
            pltpu.sync_copy(
                lw_hbm.at[pl.ds(pl.multiple_of(base * C, 128), nch0 * C)],
                lwv.at[pl.ds(0, nch0 * C)])

        @pl.when(cid == 1)
        def _():
            pltpu.sync_copy(
                lw_hbm.at[pl.ds(pl.multiple_of(base * C, 128), nch1 * C)],
                lwv.at[pl.ds(0, nch1 * C)])

        def eoff(i):
            return pl.ds(pl.multiple_of((base + i) * C, C), C)

        def fire_idx(i, q):
            pltpu.async_copy(src_hbm.at[eoff(i)], srcb[q], isem[q])
            pltpu.async_copy(dst_hbm.at[eoff(i)], dstb[q], isem[q])

        def wait_idx(q):
            pltpu.make_async_copy(src_hbm.at[pl.ds(0, C)], srcb[q], isem[q]).wait()
            pltpu.make_async_copy(dst_hbm.at[pl.ds(0, C)], dstb[q], isem[q]).wait()

        def fire_gather(q):
            pltpu.async_copy(v_hbm.at[srcb[q]], rows[q], gs[q])

        def wait_gather(q):
            pltpu.make_async_copy(v_hbm.at[srcb[0]], rows[q], gs[q]).wait()

        def fire_scatter(q):
            pltpu.async_copy(rows[q], acc.at[dstb[q]], ssem[q], add=True)

        def wait_scatter(q):
            pltpu.make_async_copy(rows[q], acc.at[dstb[q]], ssem[q]).wait()

        def scale(i, b):
            rb = rows[b]

            @plsc.parallel_loop(0, C, 1, unroll=4)
            def _(j):
                w16 = plsc.load_gather(lwv, [_splat16(i * C + j)])
                for f in range(128 // L):
                    rb[j, pl.ds(f * L, L)] = rb[j, pl.ds(f * L, L)] * w16

        with jax.named_scope("edges"):
            pltpu.sync_copy(src_hbm.at[eoff(0)], sa0)
            pltpu.sync_copy(dst_hbm.at[eoff(0)], da0)
            fire_idx(1, 1)
            fire_gather(0)

            def quad(t, _):
                for b4 in range(4):
                    i = t * 4 + b4

                    @pl.when(i + 1 < nch)
                    def _():
                        wait_idx((b4 + 1) % 4)
                        fire_gather((b4 + 1) % 4)

                    wait_gather(b4)

                    @pl.when((i + 2 < nch) & (i >= 2))
                    def _():
                        wait_scatter((b4 + 2) % 4)

                    @pl.when(i + 2 < nch)
                    def _():
                        fire_idx(i + 2, (b4 + 2) % 4)

                    scale(i, b4)
                    fire_scatter(b4)
                return 0

            lax.fori_loop(0, nch // 4, quad, 0)
            for q in range(4):
                wait_scatter(q)
            plsc.subcore_barrier()

        with jax.named_scope("drain"):
            def drain(r, _):
                off = sid * rows_per_tile + r * C
                pltpu.sync_copy(acc.at[pl.ds(off, C)], part_hbm.at[cid, pl.ds(off, C)])
                return 0

            lax.fori_loop(0, rows_per_tile // C, drain, 0)

    return prop


def _combine(parts, prev):
    npad = parts.shape[1]
    R = 512

    def body(p_ref, q_ref, o_ref):
        o_ref[...] = 2.0 * (p_ref[0] + p_ref[1]) - q_ref[...]

    return pl.pallas_call(
        body,
        grid=(npad // R,),
        in_specs=[
            pl.BlockSpec((NC, R, 128), lambda i: (0, i, 0)),
            pl.BlockSpec((R, 128), lambda i: (i, 0)),
        ],
        out_specs=pl.BlockSpec((R, 128), lambda i: (i, 0)),
        out_shape=jax.ShapeDtypeStruct((npad, 128), _F32),
    )(parts, prev)


def _combine1(parts):
    npad = parts.shape[1]
    R = 512

    def body(p_ref, o_ref):
        o_ref[...] = p_ref[0] + p_ref[1]

    return pl.pallas_call(
        body,
        grid=(npad // R,),
        in_specs=[pl.BlockSpec((NC, R, 128), lambda i: (0, i, 0))],
        out_specs=pl.BlockSpec((R, 128), lambda i: (i, 0)),
        out_shape=jax.ShapeDtypeStruct((npad, 128), _F32),
    )(parts)


def _final(xp, t1, t2, p3, Wc, bc, lin_W, lin_b):
    npad = xp.shape[0]
    R = 512

    def body(x_ref, t1_ref, t2_ref, p3_ref, w_ref, b_ref, lw_ref, lb_ref, o_ref):
        t1v = t1_ref[...]
        t3 = 2.0 * (p3_ref[0] + p3_ref[1]) - t1v
        a = jnp.dot(x_ref[...], w_ref[0], preferred_element_type=_F32)
        a = a + jnp.dot(t1v, w_ref[1], preferred_element_type=_F32)
        a = a + jnp.dot(t2_ref[...], w_ref[2], preferred_element_type=_F32)
        a = a + jnp.dot(t3, w_ref[3], preferred_element_type=_F32)
        a = a + b_ref[...]
        z = jax.nn.sigmoid(a[:, :128])
        ht = jnp.tanh(a[:, 128:])
        h = jax.nn.relu((1.0 - z) * ht)
        o_ref[...] = jnp.dot(h, lw_ref[...], preferred_element_type=_F32) + lb_ref[...]

    return pl.pallas_call(
        body,
        grid=(npad // R,),
        in_specs=[
            pl.BlockSpec((R, 128), lambda i: (i, 0)),
            pl.BlockSpec((R, 128), lambda i: (i, 0)),
            pl.BlockSpec((R, 128), lambda i: (i, 0)),
            pl.BlockSpec((NC, R, 128), lambda i: (0, i, 0)),
            pl.BlockSpec((4, 128, 256), lambda i: (0, 0, 0)),
            pl.BlockSpec((1, 256), lambda i: (0, 0)),
            pl.BlockSpec((128, 1), lambda i: (0, 0)),
            pl.BlockSpec((1, 1), lambda i: (0, 0)),
        ],
        out_specs=pl.BlockSpec((R, 1), lambda i: (i, 0)),
        out_shape=jax.ShapeDtypeStruct((npad, 1), _F32),
    )(xp, t1, t2, p3, Wc, bc, lin_W, lin_b)


def kernel(x, edge_index, edge_weight, W_xz, b_xz, W_hz, b_hz, W_xr, b_xr,
           W_hr, b_hr, W_xh, b_xh, W_hh, b_hh, lin_W, lin_b):
    n, d = x.shape
    e = edge_weight.shape[0]
    npad = _round_up(n, NS * C)
    epad = _round_up(e, NW * C * 4)
    nch = epad // (NW * C)

    src = jnp.zeros((epad,), _I32).at[:e].set(edge_index[0])
    dst = jnp.zeros((epad,), _I32).at[:e].set(edge_index[1])
    src3 = src.reshape(NW, nch, C)
    dst3 = dst.reshape(NW, nch, C)
    wgt = jnp.zeros((epad,), _F32).at[:e].set(edge_weight).reshape(NW, nch, C)
    xp = jnp.zeros((npad, d), _F32).at[:n].set(x)

    part_deg = _make_deg_partial(npad, nch)(src3, wgt)
    dis = _make_dis(npad)(part_deg)
    lw = _make_lw(npad, nch)(src3, dst3, wgt, dis)

    per_pair = epad // (NS * C)
    nch0 = (per_pair * 7 // 10) // 4 * 4
    nch1 = per_pair - nch0
    prop = _make_prop(npad, nch0, nch1)
    p1 = prop(xp, src, dst, lw)
    t1 = _combine1(p1)
    p2 = prop(t1, src, dst, lw)
    t2 = _combine(p2, xp)
    p3 = prop(t2, src, dst, lw)

    Wc = jnp.concatenate([W_xz, W_xh], axis=2)
    bc = jnp.concatenate([b_xz + b_hz, b_xh + b_hh]).reshape(1, 256)
    out = _final(xp, t1, t2, p3, Wc, bc, lin_W, lin_b.reshape(1, 1))
    return out[:n]
